# baseline (speedup 1.0000x reference)
import functools

import jax
import jax.numpy as jnp
from jax.experimental import pallas as pl
from jax.experimental.pallas import tpu as pltpu
from jax.experimental.pallas import tpu_sc as plsc

HI = jax.lax.Precision.HIGHEST
DEF = jax.lax.Precision.DEFAULT
F32 = jnp.float32


def _nt(a, b, prec=HI):
    return jax.lax.dot_general(a, b, (((1,), (1,)), ((), ())), precision=prec)


def _nn(a, b, prec=HI):
    return jax.lax.dot_general(a, b, (((1,), (0,)), ((), ())), precision=prec)


def _parity_planes(a):
    b, h, w, c = a.shape
    planes = []
    for pu in range(2):
        for pv in range(2):
            t = a.reshape(b, h // 2, 2, w, c)[:, :, pu]
            planes.append(t.reshape(b, h // 2, w // 2, 2, c)[:, :, :, pv])
    return planes


def _p1_body(x00_ref, x01_ref, x10_ref, x11_ref, w_ref, b_ref, out_ref):
    planes = ((x00_ref, x01_ref), (x10_ref, x11_ref))
    pooled = None
    for ph in range(2):
        for pw in range(2):
            acc = jnp.zeros((2048, 96), F32)
            for dy in range(5):
                for dx in range(5):
                    pu, a = (ph + dy) % 2, (ph + dy) // 2
                    pv, b = (pw + dx) % 2, (pw + dx) // 2
                    xs = planes[pu][pv][:, a:a + 16, b:b + 16, :]
                    acc = acc + _nn(xs.reshape(2048, 3), w_ref[dy, dx], DEF)
            pooled = acc if pooled is None else jnp.maximum(pooled, acc)
    p = jnp.maximum(pooled + b_ref[...], 0.0)
    p = p.reshape(8, 256, 96)
    sq = jnp.sum(p * p, axis=1, keepdims=True)
    n = jnp.maximum(jnp.sqrt(sq), 1e-12)
    out_ref[...] = p / n


def _stage1(x, conv1_w, conv1_b):
    xt = jnp.transpose(x, (0, 2, 3, 1))
    xp = jnp.pad(xt, ((0, 0), (2, 2), (2, 2), (0, 0)))
    xq = _parity_planes(xp)
    w1 = jnp.transpose(conv1_w, (2, 3, 1, 0))
    b1 = conv1_b.reshape(1, 96)
    return pl.pallas_call(
        _p1_body,
        out_shape=jax.ShapeDtypeStruct((8, 256, 96), F32),
    )(*xq, w1, b1)


def _knn_idx_body(xn_ref, cen_ref, idx_ref, *, cb, k, f):
    i = pl.program_id(0)
    cen_blk = cen_ref[...]
    censq = (cen_blk * cen_blk).reshape(cb * k, f)
    cn_mat = _nt(jnp.ones((8, f), F32), censq)
    iota = jax.lax.broadcasted_iota(jnp.int32, (8, k), 1)
    for cc in range(cb):
        xy = _nt(xn_ref[cc], cen_ref[cc])
        scores = cn_mat[:, cc * k:(cc + 1) * k] - 2.0 * xy
        m = jnp.min(scores, axis=1, keepdims=True)
        idx = jnp.min(jnp.where(scores == m, iota, k), axis=1, keepdims=True)
        idx_ref[cc] = idx + (i * cb + cc) * k


def _knn_idx(xn_t, center, cb):
    c, k, f = center.shape
    grid = c // cb
    body = functools.partial(_knn_idx_body, cb=cb, k=k, f=f)
    return pl.pallas_call(
        body,
        grid=(grid,),
        in_specs=[
            pl.BlockSpec((cb, 8, f), lambda i: (i, 0, 0)),
            pl.BlockSpec((cb, k, f), lambda i: (i, 0, 0)),
        ],
        out_specs=pl.BlockSpec((cb, 8, 1), lambda i: (i, 0, 0)),
        out_shape=jax.ShapeDtypeStruct((c, 8, 1), jnp.int32),
    )(xn_t, center)


def _sc_gather(table, idx, n, d):
    nw = 32
    bpw = n // nw
    mesh = plsc.VectorSubcoreMesh(core_axis_name="c", subcore_axis_name="s")

    @functools.partial(
        pl.kernel,
        out_type=jax.ShapeDtypeStruct((n, d), F32),
        mesh=mesh,
        scratch_types=[
            pltpu.VMEM((bpw,), jnp.int32),
            pltpu.VMEM((bpw, d), F32),
            pltpu.SemaphoreType.DMA,
        ],
    )
    def k(table_hbm, idx_hbm, out_hbm, idx_v, rows_v, sem):
        wid = jax.lax.axis_index("s") * 2 + jax.lax.axis_index("c")
        base = wid * bpw
        pltpu.sync_copy(idx_hbm.at[pl.ds(base, bpw)], idx_v)
        pltpu.async_copy(table_hbm.at[idx_v], rows_v, sem).wait()
        pltpu.sync_copy(rows_v, out_hbm.at[pl.ds(base, bpw)])

    return k(table, idx)


def _knn(xn_t, center, cb):
    c, k, f = center.shape
    idx = _knn_idx(xn_t, center, cb)
    q = _sc_gather(center.reshape(c * k, f), idx.reshape(c * 8), c * 8, f)
    return q.reshape(c, 8, f)


def _knn_oh_body(xn_ref, cen_ref, out_ref, *, cb, k, f):
    cen_blk = cen_ref[...]
    censq = (cen_blk * cen_blk).reshape(cb * k, f)
    cn_mat = _nt(jnp.ones((8, f), F32), censq)
    iota = jax.lax.broadcasted_iota(jnp.int32, (8, k), 1)
    for cc in range(cb):
        cen = cen_ref[cc]
        xy = _nt(xn_ref[cc], cen)
        scores = cn_mat[:, cc * k:(cc + 1) * k] - 2.0 * xy
        m = jnp.min(scores, axis=1, keepdims=True)
        idx = jnp.min(jnp.where(scores == m, iota, k), axis=1, keepdims=True)
        oh = (iota == idx).astype(F32)
        out_ref[cc] = _nn(oh, cen, DEF)


def _knn_oh(xn_t, center, cb):
    c, k, f = center.shape
    grid = c // cb
    body = functools.partial(_knn_oh_body, cb=cb, k=k, f=f)
    return pl.pallas_call(
        body,
        grid=(grid,),
        in_specs=[
            pl.BlockSpec((cb, 8, f), lambda i: (i, 0, 0)),
            pl.BlockSpec((cb, k, f), lambda i: (i, 0, 0)),
        ],
        out_specs=pl.BlockSpec((cb, 8, f), lambda i: (i, 0, 0)),
        out_shape=jax.ShapeDtypeStruct((c, 8, f), F32),
    )(xn_t, center)


def _p3_body(x00_ref, x01_ref, x10_ref, x11_ref, w_ref, b_ref, out_ref):
    planes = ((x00_ref, x01_ref), (x10_ref, x11_ref))
    pooled = None
    for ph in range(2):
        for pw in range(2):
            acc = jnp.zeros((512, 192), F32)
            for dy in range(5):
                for dx in range(5):
                    pu, a = (ph + dy) % 2, (ph + dy) // 2
                    pv, b = (pw + dx) % 2, (pw + dx) // 2
                    xs = planes[pu][pv][:, a:a + 8, b:b + 8, :]
                    acc = acc + _nn(xs.reshape(512, 96), w_ref[dy, dx], DEF)
            pooled = acc if pooled is None else jnp.maximum(pooled, acc)
    p = jnp.maximum(pooled + b_ref[...], 0.0)
    p = p.reshape(8, 64, 192)
    sq = jnp.sum(p * p, axis=1, keepdims=True)
    n = jnp.maximum(jnp.sqrt(sq), 1e-12)
    out_ref[...] = p / n


def _stage3(q1, conv2_w, conv2_b):
    h = jnp.transpose(q1, (1, 2, 0)).reshape(8, 16, 16, 96)
    hp = jnp.pad(h, ((0, 0), (2, 2), (2, 2), (0, 0)))
    xq = _parity_planes(hp)
    w2 = jnp.transpose(conv2_w, (2, 3, 1, 0))
    b2 = conv2_b.reshape(1, 192)
    return pl.pallas_call(
        _p3_body,
        out_shape=jax.ShapeDtypeStruct((8, 64, 192), F32),
    )(*xq, w2, b2)


def _p5_body(h_ref, w1_ref, b1_ref, w2_ref, b2_ref, out_ref):
    i = pl.program_id(0)
    t = _nt(h_ref[...], w1_ref[...], DEF)
    b = b1_ref[0, pl.ds(i * 128, 128)]
    t = jnp.maximum(t + b[None, :], 0.0)
    part = _nn(t, w2_ref[...], DEF)

    @pl.when(i == 0)
    def _():
        out_ref[...] = part + b2_ref[...]

    @pl.when(i > 0)
    def _():
        out_ref[...] = out_ref[...] + part


def _stage5(h, fc1_w, fc1_b, fc2_w, fc2_b):
    w2t = fc2_w.T
    return pl.pallas_call(
        _p5_body,
        grid=(16,),
        in_specs=[
            pl.BlockSpec((8, 12288), lambda i: (0, 0)),
            pl.BlockSpec((128, 12288), lambda i: (i, 0)),
            pl.BlockSpec((1, 2048), lambda i: (0, 0)),
            pl.BlockSpec((128, 10), lambda i: (i, 0)),
            pl.BlockSpec((1, 10), lambda i: (0, 0)),
        ],
        out_specs=pl.BlockSpec((8, 10), lambda i: (0, 0)),
        out_shape=jax.ShapeDtypeStruct((8, 10), F32),
    )(h, fc1_w, fc1_b.reshape(1, 2048), w2t, fc2_b.reshape(1, 10))


def kernel(x, conv1_w, conv1_b, conv2_w, conv2_b, fc1_w, fc1_b, fc2_w, fc2_b,
           center0, center1):
    xn1 = _stage1(x, conv1_w, conv1_b)
    q1 = _knn(jnp.transpose(xn1, (2, 0, 1)), center0, cb=8)
    xn2 = _stage3(q1, conv2_w, conv2_b)
    q2 = _knn_oh(jnp.transpose(xn2, (2, 0, 1)), center1, cb=16)
    h = jnp.transpose(q2, (1, 0, 2)).reshape(8, 12288)
    return _stage5(h, fc1_w, fc1_b, fc2_w, fc2_b)


# speedup vs baseline: 1.4382x; 1.4382x over previous
import functools

import jax
import jax.numpy as jnp
from jax.experimental import pallas as pl
from jax.experimental.pallas import tpu as pltpu
from jax.experimental.pallas import tpu_sc as plsc

HI = jax.lax.Precision.HIGHEST
DEF = jax.lax.Precision.DEFAULT
F32 = jnp.float32


def _nt(a, b, prec=HI):
    return jax.lax.dot_general(a, b, (((1,), (1,)), ((), ())), precision=prec)


def _nn(a, b, prec=HI):
    return jax.lax.dot_general(a, b, (((1,), (0,)), ((), ())), precision=prec)


def _parity_planes(a):
    b, h, w, c = a.shape
    planes = []
    for pu in range(2):
        for pv in range(2):
            t = a.reshape(b, h // 2, 2, w, c)[:, :, pu]
            planes.append(t.reshape(b, h // 2, w // 2, 2, c)[:, :, :, pv])
    return planes


def _p1_body(x00_ref, x01_ref, x10_ref, x11_ref, w_ref, b_ref, out_ref):
    planes = ((x00_ref, x01_ref), (x10_ref, x11_ref))
    pooled = None
    for ph in range(2):
        for pw in range(2):
            acc = jnp.zeros((2048, 96), F32)
            for dy in range(5):
                for dx in range(5):
                    pu, a = (ph + dy) % 2, (ph + dy) // 2
                    pv, b = (pw + dx) % 2, (pw + dx) // 2
                    xs = planes[pu][pv][:, a:a + 16, b:b + 16, :]
                    acc = acc + _nn(xs.reshape(2048, 3), w_ref[dy, dx], DEF)
            pooled = acc if pooled is None else jnp.maximum(pooled, acc)
    p = jnp.maximum(pooled + b_ref[...], 0.0)
    p = p.reshape(8, 256, 96)
    sq = jnp.sum(p * p, axis=1, keepdims=True)
    n = jnp.maximum(jnp.sqrt(sq), 1e-12)
    out_ref[...] = p / n


def _stage1(x, conv1_w, conv1_b):
    xt = jnp.transpose(x, (0, 2, 3, 1))
    xp = jnp.pad(xt, ((0, 0), (2, 2), (2, 2), (0, 0)))
    xq = _parity_planes(xp)
    w1 = jnp.transpose(conv1_w, (2, 3, 1, 0))
    b1 = conv1_b.reshape(1, 96)
    return pl.pallas_call(
        _p1_body,
        out_shape=jax.ShapeDtypeStruct((8, 256, 96), F32),
    )(*xq, w1, b1)


def _aug_scores(xc, cen, k, f):
    cen2 = jnp.sum(cen * cen, axis=1, keepdims=True)
    lhs = jnp.concatenate([-2.0 * xc, jnp.ones((8, 1), F32)], axis=1)
    rhs = jnp.concatenate([cen, cen2], axis=1)
    return _nt(lhs, rhs)


def _knn_idx_body(xn_ref, cen_ref, idx_ref, *, cb, k, f):
    i = pl.program_id(0)
    iota = jax.lax.broadcasted_iota(jnp.int32, (8, k), 1)
    for cc in range(cb):
        scores = _aug_scores(xn_ref[cc], cen_ref[cc], k, f)
        m = jnp.min(scores, axis=1, keepdims=True)
        idx = jnp.min(jnp.where(scores == m, iota, k), axis=1, keepdims=True)
        idx_ref[cc] = idx + (i * cb + cc) * k


def _knn_idx(xn_t, center, cb):
    c, k, f = center.shape
    grid = c // cb
    body = functools.partial(_knn_idx_body, cb=cb, k=k, f=f)
    return pl.pallas_call(
        body,
        grid=(grid,),
        in_specs=[
            pl.BlockSpec((cb, 8, f), lambda i: (i, 0, 0)),
            pl.BlockSpec((cb, k, f), lambda i: (i, 0, 0)),
        ],
        out_specs=pl.BlockSpec((cb, 8, 1), lambda i: (i, 0, 0)),
        out_shape=jax.ShapeDtypeStruct((c, 8, 1), jnp.int32),
    )(xn_t, center)


def _sc_gather(table, idx, n, d):
    nw = 32
    bpw = n // nw
    mesh = plsc.VectorSubcoreMesh(core_axis_name="c", subcore_axis_name="s")

    @functools.partial(
        pl.kernel,
        out_type=jax.ShapeDtypeStruct((n, d), F32),
        mesh=mesh,
        scratch_types=[
            pltpu.VMEM((bpw,), jnp.int32),
            pltpu.VMEM((bpw, d), F32),
            pltpu.SemaphoreType.DMA,
        ],
    )
    def k(table_hbm, idx_hbm, out_hbm, idx_v, rows_v, sem):
        wid = jax.lax.axis_index("s") * 2 + jax.lax.axis_index("c")
        base = wid * bpw
        pltpu.sync_copy(idx_hbm.at[pl.ds(base, bpw)], idx_v)
        pltpu.async_copy(table_hbm.at[idx_v], rows_v, sem).wait()
        pltpu.sync_copy(rows_v, out_hbm.at[pl.ds(base, bpw)])

    return k(table, idx)


def _knn(xn_t, center, cb):
    c, k, f = center.shape
    idx = _knn_idx(xn_t, center, cb)
    q = _sc_gather(center.reshape(c * k, f), idx.reshape(c * 8), c * 8, f)
    return q.reshape(c, 8, f)


def _knn_oh_body(xn_ref, cen_ref, out_ref, *, cb, k, f):
    iota = jax.lax.broadcasted_iota(jnp.int32, (8, k), 1)
    for cc in range(cb):
        cen = cen_ref[cc]
        scores = _aug_scores(xn_ref[cc], cen, k, f)
        m = jnp.min(scores, axis=1, keepdims=True)
        idx = jnp.min(jnp.where(scores == m, iota, k), axis=1, keepdims=True)
        for b in range(8):
            s = idx[b, 0]
            out_ref[cc, b:b + 1, :] = cen_ref[cc, pl.ds(s, 1), :]


def _knn_oh(xn_t, center, cb):
    c, k, f = center.shape
    grid = c // cb
    body = functools.partial(_knn_oh_body, cb=cb, k=k, f=f)
    return pl.pallas_call(
        body,
        grid=(grid,),
        in_specs=[
            pl.BlockSpec((cb, 8, f), lambda i: (i, 0, 0)),
            pl.BlockSpec((cb, k, f), lambda i: (i, 0, 0)),
        ],
        out_specs=pl.BlockSpec((cb, 8, f), lambda i: (i, 0, 0)),
        out_shape=jax.ShapeDtypeStruct((c, 8, f), F32),
    )(xn_t, center)


def _p3_body(x00_ref, x01_ref, x10_ref, x11_ref, w_ref, b_ref, out_ref):
    planes = ((x00_ref, x01_ref), (x10_ref, x11_ref))
    pooled = None
    for ph in range(2):
        for pw in range(2):
            acc = jnp.zeros((512, 192), F32)
            for dy in range(5):
                for dx in range(5):
                    pu, a = (ph + dy) % 2, (ph + dy) // 2
                    pv, b = (pw + dx) % 2, (pw + dx) // 2
                    xs = planes[pu][pv][:, a:a + 8, b:b + 8, :]
                    acc = acc + _nn(xs.reshape(512, 96), w_ref[dy, dx], DEF)
            pooled = acc if pooled is None else jnp.maximum(pooled, acc)
    p = jnp.maximum(pooled + b_ref[...], 0.0)
    p = p.reshape(8, 64, 192)
    sq = jnp.sum(p * p, axis=1, keepdims=True)
    n = jnp.maximum(jnp.sqrt(sq), 1e-12)
    out_ref[...] = p / n


def _stage3(q1, conv2_w, conv2_b):
    h = jnp.transpose(q1, (1, 2, 0)).reshape(8, 16, 16, 96)
    hp = jnp.pad(h, ((0, 0), (2, 2), (2, 2), (0, 0)))
    xq = _parity_planes(hp)
    w2 = jnp.transpose(conv2_w, (2, 3, 1, 0))
    b2 = conv2_b.reshape(1, 192)
    return pl.pallas_call(
        _p3_body,
        out_shape=jax.ShapeDtypeStruct((8, 64, 192), F32),
    )(*xq, w2, b2)


def _p5_body(h_ref, w1_ref, b1_ref, w2_ref, b2_ref, out_ref):
    i = pl.program_id(0)
    t = _nt(h_ref[...], w1_ref[...], DEF)
    b = b1_ref[0, pl.ds(i * 128, 128)]
    t = jnp.maximum(t + b[None, :], 0.0)
    part = _nn(t, w2_ref[...], DEF)

    @pl.when(i == 0)
    def _():
        out_ref[...] = part + b2_ref[...]

    @pl.when(i > 0)
    def _():
        out_ref[...] = out_ref[...] + part


def _stage5(h, fc1_w, fc1_b, fc2_w, fc2_b):
    w2t = fc2_w.T
    return pl.pallas_call(
        _p5_body,
        grid=(16,),
        in_specs=[
            pl.BlockSpec((8, 12288), lambda i: (0, 0)),
            pl.BlockSpec((128, 12288), lambda i: (i, 0)),
            pl.BlockSpec((1, 2048), lambda i: (0, 0)),
            pl.BlockSpec((128, 10), lambda i: (i, 0)),
            pl.BlockSpec((1, 10), lambda i: (0, 0)),
        ],
        out_specs=pl.BlockSpec((8, 10), lambda i: (0, 0)),
        out_shape=jax.ShapeDtypeStruct((8, 10), F32),
    )(h, fc1_w, fc1_b.reshape(1, 2048), w2t, fc2_b.reshape(1, 10))


def kernel(x, conv1_w, conv1_b, conv2_w, conv2_b, fc1_w, fc1_b, fc2_w, fc2_b,
           center0, center1):
    xn1 = _stage1(x, conv1_w, conv1_b)
    q1 = _knn(jnp.transpose(xn1, (2, 0, 1)), center0, cb=8)
    xn2 = _stage3(q1, conv2_w, conv2_b)
    q2 = _knn_oh(jnp.transpose(xn2, (2, 0, 1)), center1, cb=16)
    h = jnp.transpose(q2, (1, 0, 2)).reshape(8, 12288)
    return _stage5(h, fc1_w, fc1_b, fc2_w, fc2_b)
